# baseline (speedup 1.0000x reference)
import functools

import jax
import jax.numpy as jnp
from jax import lax
from jax.experimental import pallas as pl
from jax.experimental.pallas import tpu as pltpu
from jax.experimental.pallas import tpu_sc as plsc

B = 4096
V = 100000
NS = 26
EMB = 32
SEQ = 50
DD = 13
H = 200

NC = 2
NSUB = 16
NW = NC * NSUB
SPW = B // NW
WEPC = SPW * NS
GW = 32
NG = WEPC // GW
PAIRS = SPW // 2
NSP = 32


def _mesh():
    return plsc.VectorSubcoreMesh(core_axis_name="c", subcore_axis_name="s",
                                  num_cores=NC, num_subcores=NSUB)


def _wid():
    return lax.axis_index("s") * NC + lax.axis_index("c")


VCH = 128
NVCH = V // VCH
VTAIL = V - NVCH * VCH


def _tr_body(embT_hbm, tail_hbm, embrm_hbm, slab0, slab1, outslab,
             isem0, isem1, osem):
    wid = _wid()
    slabs = (slab0, slab1)
    isems = (isem0, isem1)
    iota16 = lax.iota(jnp.int32, 16)

    def _transpose(slab, width, out):
        def _vbody(v2, _):
            r4 = lax.shift_right_logical(v2, 2)
            cc = lax.shift_left(v2 & 3, 5)
            a = plsc.load_gather(slab, [iota16, jnp.full((16,), v2,
                                                         jnp.int32)])
            bvals = plsc.load_gather(slab, [iota16 + 16,
                                            jnp.full((16,), v2, jnp.int32)])
            out[r4, pl.ds(cc, 16)] = a
            out[r4, pl.ds(cc + 16, 16)] = bvals
            return 0
        lax.fori_loop(0, width, _vbody, 0)

    def _fbody(f, _):
        n_c = lax.shift_right_logical(NVCH - wid + NW - 1, 5)

        def _cpair(j2, _):
            for b in range(2):
                j = 2 * j2 + b

                @pl.when(j < n_c)
                def _():
                    c = wid + lax.shift_left(j, 5)
                    nxt_c = c + NW

                    @pl.when(j == 0)
                    def _():
                        voff = pl.multiple_of(c * VCH, VCH)
                        pltpu.async_copy(
                            embT_hbm.at[f, :, pl.ds(voff, VCH)],
                            slabs[b], isems[b])

                    @pl.when(j + 1 < n_c)
                    def _():
                        voff2 = pl.multiple_of(nxt_c * VCH, VCH)
                        pltpu.async_copy(
                            embT_hbm.at[f, :, pl.ds(voff2, VCH)],
                            slabs[1 - b], isems[1 - b])
                    pltpu.make_async_copy(
                        embT_hbm.at[0, :, pl.ds(0, VCH)], slabs[b],
                        isems[b]).wait()
                    _transpose(slabs[b], VCH, outslab)
                    goff = pl.multiple_of(
                        f * (V // 4) + lax.shift_left(c, 5), 8)
                    pltpu.sync_copy(outslab,
                                    embrm_hbm.at[pl.ds(goff, 32), :])
            return 0

        lax.fori_loop(0, (NVCH // NW + 2) // 2, _cpair, 0)
        return 0

    lax.fori_loop(0, NS, _fbody, 0)

    @pl.when(wid < NS)
    def _():
        toff = pl.multiple_of(wid * (VTAIL // 4), 8)
        goff = pl.multiple_of(wid * (V // 4) + (NVCH * VCH) // 4, 8)
        pltpu.sync_copy(tail_hbm.at[pl.ds(toff, VTAIL // 4), :],
                        embrm_hbm.at[pl.ds(goff, VTAIL // 4), :])


@functools.lru_cache(maxsize=1)
def _make_tr_call():
    return pl.kernel(
        _tr_body,
        out_type=(jax.ShapeDtypeStruct((NS * V // 4, 128), jnp.float32),),
        mesh=_mesh(),
        scratch_types=[
            pltpu.VMEM((EMB, VCH), jnp.float32),
            pltpu.VMEM((EMB, VCH), jnp.float32),
            pltpu.VMEM((32, 128), jnp.float32),
            pltpu.SemaphoreType.DMA,
            pltpu.SemaphoreType.DMA,
            pltpu.SemaphoreType.DMA,
        ],
        compiler_params=pltpu.CompilerParams(use_tc_tiling_on_sc=True,
                                             needs_layout_passes=False),
    )


NT2 = WEPC // 128


def _se_body(embrm_hbm, sp_hbm, foffs_hbm,
             embout_hbm,
             spv, fov, g2d, w2d, estage, outv0, outv1,
             gsem0, gsem1, ovsem0, ovsem1):
    gsems = (gsem0, gsem1)
    ovsems = (ovsem0, ovsem1)
    outvs = (outv0, outv1)
    base_e = pl.multiple_of(_wid() * WEPC, WEPC)
    pltpu.sync_copy(sp_hbm.at[pl.ds(base_e, WEPC)], spv)
    pltpu.sync_copy(foffs_hbm, fov)

    def _jbody(j, _):
        def _lbody(l, _):
            off = lax.shift_left(j, 7) + lax.shift_left(l, 4)
            flat = spv[pl.ds(off, 16)] + fov[pl.ds(off, 16)]
            g2d[j, pl.ds(l * 16, 16)] = lax.shift_right_logical(flat, 2)
            w2d[j, pl.ds(l * 16, 16)] = lax.shift_left(flat & 3, 5)
            return 0
        return lax.fori_loop(0, 8, _lbody, 0)

    lax.fori_loop(0, NT2, _jbody, 0)

    pltpu.async_copy(embrm_hbm.at[g2d.at[0]], estage.at[0], gsems[0])

    def _jpair(h, _):
        for b in range(2):
            j = 2 * h + b

            @pl.when(j + 1 < NT2)
            def _():
                pltpu.async_copy(embrm_hbm.at[g2d.at[j + 1]],
                                 estage.at[1 - b], gsems[1 - b])
            pltpu.make_async_copy(embrm_hbm.at[g2d.at[0]], estage.at[b],
                                  gsems[b]).wait()

            @pl.when(j >= 2)
            def _():
                pltpu.make_async_copy(
                    outvs[b], embout_hbm.at[pl.ds(0, 128 * EMB)],
                    ovsems[b]).wait()

            def _ex(l, _, b=b, j=j):
                w16 = w2d[j, pl.ds(l * 16, 16)]
                for i in range(16):
                    e = l * 16 + i
                    woff = w16[i]
                    outvs[b][pl.ds(e * 32, 16)] = (
                        estage[b, e, pl.ds(woff, 16)])
                    outvs[b][pl.ds(e * 32 + 16, 16)] = (
                        estage[b, e, pl.ds(woff + 16, 16)])
                return 0

            lax.fori_loop(0, 8, _ex, 0)
            dst = pl.multiple_of((base_e + lax.shift_left(j, 7)) * EMB,
                                 128 * EMB)
            pltpu.async_copy(outvs[b],
                             embout_hbm.at[pl.ds(dst, 128 * EMB)], ovsems[b])
        return 0

    lax.fori_loop(0, NT2 // 2, _jpair, 0)
    for b in range(2):
        pltpu.make_async_copy(outvs[b], embout_hbm.at[pl.ds(0, 128 * EMB)],
                              ovsems[b]).wait()


@functools.lru_cache(maxsize=1)
def _make_se_call():
    return pl.kernel(
        _se_body,
        out_type=(jax.ShapeDtypeStruct((B * NS * EMB,), jnp.float32),),
        mesh=_mesh(),
        scratch_types=[
            pltpu.VMEM((WEPC,), jnp.int32),
            pltpu.VMEM((WEPC,), jnp.int32),
            pltpu.VMEM((NT2, 128), jnp.int32),
            pltpu.VMEM((NT2, 128), jnp.int32),
            pltpu.VMEM((2, 128, 128), jnp.float32),
            pltpu.VMEM((128 * EMB,), jnp.float32),
            pltpu.VMEM((128 * EMB,), jnp.float32),
            pltpu.SemaphoreType.DMA,
            pltpu.SemaphoreType.DMA,
            pltpu.SemaphoreType.DMA,
            pltpu.SemaphoreType.DMA,
        ],
        compiler_params=pltpu.CompilerParams(use_tc_tiling_on_sc=True),
    )


def _sl_body(lin_hbm, st0_hbm, st1_hbm, spt_hbm, s0_hbm, s1_hbm,
             seqout_hbm, linout_hbm,
             sptv, linidx, linval, linout_v, sidx0, sidx1,
             stage0, stage1, seqacc,
             lsem, s0sem, s1sem):
    wid = _wid()
    cb = wid * SPW
    zero = jnp.zeros((16,), jnp.float32)
    spt_off = pl.multiple_of(wid * NSP, NSP)
    pair_off = pl.multiple_of(cb // 2, PAIRS)
    ch_off = pl.multiple_of(cb, SPW)
    pltpu.sync_copy(spt_hbm.at[pl.ds(spt_off, NSP), :], sptv)
    pltpu.sync_copy(s0_hbm.at[pl.ds(pair_off, PAIRS), :], sidx0)
    pltpu.sync_copy(s1_hbm.at[pl.ds(pair_off, PAIRS), :], sidx1)
    for f in range(NS):
        for l in range(SPW // 16):
            linidx[f, pl.ds(l * 16, 16)] = (
                sptv[f, pl.ds(l * 16, 16)] + jnp.int32(f * V))
    lcps = [
        pltpu.async_copy(lin_hbm.at[linidx.at[f]], linval.at[f], lsem)
        for f in range(NS)
    ]
    def _pbody(p, _):
        cp0 = pltpu.async_copy(st0_hbm.at[sidx0.at[p]], stage0, s0sem)
        cp1 = pltpu.async_copy(st1_hbm.at[sidx1.at[p]], stage1, s1sem)
        cp0.wait()
        cp1.wait()
        for k in range(2):
            def _rbody(r, carry, k=k):
                a0, b0, a1, b1 = carry
                row = k * SEQ + r
                return (a0 + stage0[row, pl.ds(0, 16)],
                        b0 + stage0[row, pl.ds(16, 16)],
                        a1 + stage1[row, pl.ds(0, 16)],
                        b1 + stage1[row, pl.ds(16, 16)])
            a0, b0, a1, b1 = lax.fori_loop(
                0, SEQ, _rbody, (zero, zero, zero, zero))
            s = 2 * p + k
            seqacc[s, pl.ds(0, 16)] = a0
            seqacc[s, pl.ds(16, 16)] = b0
            seqacc[s, pl.ds(32, 16)] = a1
            seqacc[s, pl.ds(48, 16)] = b1
        return 0

    lax.fori_loop(0, PAIRS, _pbody, 0)
    pltpu.sync_copy(seqacc, seqout_hbm.at[pl.ds(ch_off, SPW), :])
    for cp in lcps:
        cp.wait()
    for l in range(SPW // 16):
        def _fbody(f, acc, l=l):
            return acc + linval[f, pl.ds(l * 16, 16)]
        linout_v[pl.ds(l * 16, 16)] = lax.fori_loop(0, NS, _fbody, zero)
    pltpu.sync_copy(linout_v, linout_hbm.at[pl.ds(ch_off, SPW)])


@functools.lru_cache(maxsize=1)
def _make_sl_call():
    return pl.kernel(
        _sl_body,
        out_type=(
            jax.ShapeDtypeStruct((B, 2 * EMB), jnp.float32),
            jax.ShapeDtypeStruct((B,), jnp.float32),
        ),
        mesh=_mesh(),
        scratch_types=[
            pltpu.VMEM((NSP, SPW), jnp.int32),
            pltpu.VMEM((NS, SPW), jnp.int32),
            pltpu.VMEM((NS, SPW), jnp.float32),
            pltpu.VMEM((SPW,), jnp.float32),
            pltpu.VMEM((PAIRS, 2 * SEQ), jnp.int32),
            pltpu.VMEM((PAIRS, 2 * SEQ), jnp.int32),
            pltpu.VMEM((2 * SEQ, EMB), jnp.float32),
            pltpu.VMEM((2 * SEQ, EMB), jnp.float32),
            pltpu.VMEM((SPW, 2 * EMB), jnp.float32),
            pltpu.SemaphoreType.DMA,
            pltpu.SemaphoreType.DMA,
            pltpu.SemaphoreType.DMA,
        ],
        compiler_params=pltpu.CompilerParams(use_tc_tiling_on_sc=False),
    )


BB = 512


def _mlp_body(dense, emb, seqp, lin, w1d, w1e, w1s, b1, w2, b2, w3, b3,
              w4, b4, wlin, blin, wf, bf, wl, bl, fin, like):
    x = jnp.dot(emb[...], w1e[...], preferred_element_type=jnp.float32)
    x = x + jnp.dot(dense[...], w1d[...], preferred_element_type=jnp.float32)
    x = x + jnp.dot(seqp[...], w1s[...], preferred_element_type=jnp.float32)
    h = jnp.maximum(x + b1[...], 0.0)
    h = jnp.maximum(
        jnp.dot(h, w2[...], preferred_element_type=jnp.float32) + b2[...], 0.0)
    h = jnp.maximum(
        jnp.dot(h, w3[...], preferred_element_type=jnp.float32) + b3[...], 0.0)
    dnn = jnp.sum(h * w4[...], axis=1, keepdims=True) + b4[0]
    first = jnp.sum(dense[...] * wlin[...], axis=1, keepdims=True) + blin[0] + lin[...]
    logits = first + dnn
    fin[...] = jax.nn.sigmoid(logits * wf[0, 0] + bf[0])
    like[...] = jax.nn.sigmoid(logits * wl[0, 0] + bl[0])


def _full(shape):
    nd = len(shape)
    return pl.BlockSpec(shape, lambda i, nd=nd: (0,) * nd)


_mlp_call = pl.pallas_call(
    _mlp_body,
    grid=(B // BB,),
    in_specs=[
        pl.BlockSpec((BB, DD), lambda i: (i, 0)),
        pl.BlockSpec((BB, NS * EMB), lambda i: (i, 0)),
        pl.BlockSpec((BB, 2 * EMB), lambda i: (i, 0)),
        pl.BlockSpec((BB, 1), lambda i: (i, 0)),
        _full((DD, H)),
        _full((NS * EMB, H)),
        _full((2 * EMB, H)),
        _full((H,)),
        _full((H, H)),
        _full((H,)),
        _full((H, H)),
        _full((H,)),
        _full((1, H)),
        _full((1,)),
        _full((1, DD)),
        _full((1,)),
        _full((1, 1)),
        _full((1,)),
        _full((1, 1)),
        _full((1,)),
    ],
    out_specs=[
        pl.BlockSpec((BB, 1), lambda i: (i, 0)),
        pl.BlockSpec((BB, 1), lambda i: (i, 0)),
    ],
    out_shape=[
        jax.ShapeDtypeStruct((B, 1), jnp.float32),
        jax.ShapeDtypeStruct((B, 1), jnp.float32),
    ],
)


def kernel(sparse_inputs, dense_inputs, seq_inputs_0, seq_inputs_1,
           lin_tables, emb_tables, seq_table_0, seq_table_1,
           W_lin, b_lin, W1, b1, W2, b2, W3, b3, W4, b4, Wf, bf, Wl, bl):
    sp = sparse_inputs.astype(jnp.int32)
    lin_flat = lin_tables.reshape(NS * V)
    sp_flat = sp.reshape(B * NS)
    spt = jnp.pad(sp.T.reshape(NS, B // SPW, SPW).transpose(1, 0, 2),
                  ((0, 0), (0, NSP - NS), (0, 0))).reshape(
        (B // SPW) * NSP, SPW)
    s0r = seq_inputs_0.astype(jnp.int32).reshape(B // 2, 2 * SEQ)
    s1r = seq_inputs_1.astype(jnp.int32).reshape(B // 2, 2 * SEQ)
    foffs = (jnp.arange(WEPC, dtype=jnp.int32) % NS) * V
    embT = jnp.transpose(emb_tables, (0, 2, 1))
    tail_rm = emb_tables[:, NVCH * VCH:, :].reshape(NS * VTAIL // 4, 128)

    seqout, linout = _make_sl_call()(
        lin_flat, seq_table_0, seq_table_1, spt, s0r, s1r)
    (embrm,) = _make_tr_call()(embT, tail_rm)
    (embout,) = _make_se_call()(embrm, sp_flat, foffs)

    W1d = W1[:DD]
    W1e = W1[DD:DD + NS * EMB]
    W1s = W1[DD + NS * EMB:] * jnp.float32(1.0 / SEQ)
    fin, like = _mlp_call(
        dense_inputs, embout.reshape(B, NS * EMB), seqout,
        linout.reshape(B, 1),
        W1d, W1e, W1s, b1, W2, b2, W3, b3,
        W4.reshape(1, H), b4, W_lin.reshape(1, DD), b_lin, Wf, bf, Wl, bl)
    return (fin, like)


# speedup vs baseline: 1.9013x; 1.9013x over previous
import functools

import jax
import jax.numpy as jnp
from jax import lax
from jax.experimental import pallas as pl
from jax.experimental.pallas import tpu as pltpu
from jax.experimental.pallas import tpu_sc as plsc

B = 4096
V = 100000
NS = 26
EMB = 32
SEQ = 50
DD = 13
H = 200

NC = 2
NSUB = 16
NW = NC * NSUB
SPW = B // NW
WEPC = SPW * NS
GW = 32
NG = WEPC // GW
PAIRS = SPW // 2
NSP = 32


def _mesh():
    return plsc.VectorSubcoreMesh(core_axis_name="c", subcore_axis_name="s",
                                  num_cores=NC, num_subcores=NSUB)


def _wid():
    return lax.axis_index("s") * NC + lax.axis_index("c")


def _se_body(emb_hbm, sp_hbm, fidx_hbm,
             embout_hbm,
             spv, fxv, stage, outv0, outv1,
             gsem0, gsem1, ovsem0, ovsem1):
    gsems = (gsem0, gsem1)
    ovsems = (ovsem0, ovsem1)
    outvs = (outv0, outv1)
    base_e = pl.multiple_of(_wid() * WEPC, WEPC)
    pltpu.sync_copy(sp_hbm.at[pl.ds(base_e, WEPC)], spv)
    pltpu.sync_copy(fidx_hbm, fxv)

    def _fire(g, b):
        for l in range(GW // 16):
            v16 = spv[pl.ds(g * GW + l * 16, 16)]
            f16 = fxv[pl.ds(g * GW + l * 16, 16)]
            blk16 = lax.shift_left(lax.shift_right_logical(v16, 3), 3)
            for i in range(16):
                vb = pl.multiple_of(blk16[i], 8)
                pltpu.async_copy(emb_hbm.at[f16[i], pl.ds(vb, 8), :],
                                 stage.at[b, l * 16 + i], gsems[b])

    def _drain_gather(b):
        for _ in range(GW):
            pltpu.make_async_copy(emb_hbm.at[0, pl.ds(0, 8), :],
                                  stage.at[b, 0], gsems[b]).wait()

    def _extract(g, b):
        for l in range(GW // 16):
            v16 = spv[pl.ds(g * GW + l * 16, 16)]
            for i in range(16):
                r = v16[i] & 7
                e = l * 16 + i
                outvs[b][pl.ds(e * 32, 16)] = stage[b, e, r, pl.ds(0, 16)]
                outvs[b][pl.ds(e * 32 + 16, 16)] = (
                    stage[b, e, r, pl.ds(16, 16)])

    def _writeback(g, b):
        dst = pl.multiple_of((base_e + g * GW) * EMB, GW * EMB)
        pltpu.async_copy(outvs[b],
                         embout_hbm.at[pl.ds(dst, GW * EMB)], ovsems[b])

    def _drain_wb(b):
        pltpu.make_async_copy(outvs[b], embout_hbm.at[pl.ds(0, GW * EMB)],
                              ovsems[b]).wait()

    _fire(0, 0)

    def _h_body(h, _):
        for b in range(2):
            g = 2 * h + b
            nxt = g + 1

            @pl.when(nxt < NG)
            def _():
                _fire(nxt, 1 - b)
            _drain_gather(b)

            @pl.when(g >= 2)
            def _():
                _drain_wb(b)
            _extract(g, b)
            _writeback(g, b)
        return 0

    lax.fori_loop(0, NG // 2, _h_body, 0)
    _drain_wb(0)
    _drain_wb(1)


@functools.lru_cache(maxsize=1)
def _make_se_call():
    return pl.kernel(
        _se_body,
        out_type=(jax.ShapeDtypeStruct((B * NS * EMB,), jnp.float32),),
        mesh=_mesh(),
        scratch_types=[
            pltpu.VMEM((WEPC,), jnp.int32),
            pltpu.VMEM((WEPC,), jnp.int32),
            pltpu.VMEM((2, GW, 8, EMB), jnp.float32),
            pltpu.VMEM((GW * EMB,), jnp.float32),
            pltpu.VMEM((GW * EMB,), jnp.float32),
            pltpu.SemaphoreType.DMA,
            pltpu.SemaphoreType.DMA,
            pltpu.SemaphoreType.DMA,
            pltpu.SemaphoreType.DMA,
        ],
        compiler_params=pltpu.CompilerParams(use_tc_tiling_on_sc=True),
    )


def _sl_body(lin_hbm, st0_hbm, st1_hbm, spt_hbm, s0_hbm, s1_hbm,
             seqout_hbm, linout_hbm,
             sptv, linidx, linval, linout_v, sidx0, sidx1,
             stage0, stage1, seqacc,
             lsem, s0sem, s1sem):
    wid = _wid()
    cb = wid * SPW
    zero = jnp.zeros((16,), jnp.float32)
    spt_off = pl.multiple_of(wid * NSP, NSP)
    pair_off = pl.multiple_of(cb // 2, PAIRS)
    ch_off = pl.multiple_of(cb, SPW)
    pltpu.sync_copy(spt_hbm.at[pl.ds(spt_off, NSP), :], sptv)
    pltpu.sync_copy(s0_hbm.at[pl.ds(pair_off, PAIRS), :], sidx0)
    pltpu.sync_copy(s1_hbm.at[pl.ds(pair_off, PAIRS), :], sidx1)
    for f in range(NS):
        for l in range(SPW // 16):
            linidx[f, pl.ds(l * 16, 16)] = (
                sptv[f, pl.ds(l * 16, 16)] + jnp.int32(f * V))
    lcps = [
        pltpu.async_copy(lin_hbm.at[linidx.at[f]], linval.at[f], lsem)
        for f in range(NS)
    ]
    def _pbody(p, _):
        cp0 = pltpu.async_copy(st0_hbm.at[sidx0.at[p]], stage0, s0sem)
        cp1 = pltpu.async_copy(st1_hbm.at[sidx1.at[p]], stage1, s1sem)
        cp0.wait()
        cp1.wait()
        for k in range(2):
            def _rbody(r, carry, k=k):
                a0, b0, a1, b1 = carry
                row = k * SEQ + r
                return (a0 + stage0[row, pl.ds(0, 16)],
                        b0 + stage0[row, pl.ds(16, 16)],
                        a1 + stage1[row, pl.ds(0, 16)],
                        b1 + stage1[row, pl.ds(16, 16)])
            a0, b0, a1, b1 = lax.fori_loop(
                0, SEQ, _rbody, (zero, zero, zero, zero))
            s = 2 * p + k
            seqacc[s, pl.ds(0, 16)] = a0
            seqacc[s, pl.ds(16, 16)] = b0
            seqacc[s, pl.ds(32, 16)] = a1
            seqacc[s, pl.ds(48, 16)] = b1
        return 0

    lax.fori_loop(0, PAIRS, _pbody, 0)
    pltpu.sync_copy(seqacc, seqout_hbm.at[pl.ds(ch_off, SPW), :])
    for cp in lcps:
        cp.wait()
    for l in range(SPW // 16):
        def _fbody(f, acc, l=l):
            return acc + linval[f, pl.ds(l * 16, 16)]
        linout_v[pl.ds(l * 16, 16)] = lax.fori_loop(0, NS, _fbody, zero)
    pltpu.sync_copy(linout_v, linout_hbm.at[pl.ds(ch_off, SPW)])


@functools.lru_cache(maxsize=1)
def _make_sl_call():
    return pl.kernel(
        _sl_body,
        out_type=(
            jax.ShapeDtypeStruct((B, 2 * EMB), jnp.float32),
            jax.ShapeDtypeStruct((B,), jnp.float32),
        ),
        mesh=_mesh(),
        scratch_types=[
            pltpu.VMEM((NSP, SPW), jnp.int32),
            pltpu.VMEM((NS, SPW), jnp.int32),
            pltpu.VMEM((NS, SPW), jnp.float32),
            pltpu.VMEM((SPW,), jnp.float32),
            pltpu.VMEM((PAIRS, 2 * SEQ), jnp.int32),
            pltpu.VMEM((PAIRS, 2 * SEQ), jnp.int32),
            pltpu.VMEM((2 * SEQ, EMB), jnp.float32),
            pltpu.VMEM((2 * SEQ, EMB), jnp.float32),
            pltpu.VMEM((SPW, 2 * EMB), jnp.float32),
            pltpu.SemaphoreType.DMA,
            pltpu.SemaphoreType.DMA,
            pltpu.SemaphoreType.DMA,
        ],
        compiler_params=pltpu.CompilerParams(use_tc_tiling_on_sc=False),
    )


BB = 512


def _mlp_body(dense, emb, seqp, lin, w1d, w1e, w1s, b1, w2, b2, w3, b3,
              w4, b4, wlin, blin, wf, bf, wl, bl, fin, like):
    x = jnp.dot(emb[...], w1e[...], preferred_element_type=jnp.float32)
    x = x + jnp.dot(dense[...], w1d[...], preferred_element_type=jnp.float32)
    x = x + jnp.dot(seqp[...], w1s[...], preferred_element_type=jnp.float32)
    h = jnp.maximum(x + b1[...], 0.0)
    h = jnp.maximum(
        jnp.dot(h, w2[...], preferred_element_type=jnp.float32) + b2[...], 0.0)
    h = jnp.maximum(
        jnp.dot(h, w3[...], preferred_element_type=jnp.float32) + b3[...], 0.0)
    dnn = jnp.sum(h * w4[...], axis=1, keepdims=True) + b4[0]
    first = jnp.sum(dense[...] * wlin[...], axis=1, keepdims=True) + blin[0] + lin[...]
    logits = first + dnn
    fin[...] = jax.nn.sigmoid(logits * wf[0, 0] + bf[0])
    like[...] = jax.nn.sigmoid(logits * wl[0, 0] + bl[0])


def _full(shape):
    nd = len(shape)
    return pl.BlockSpec(shape, lambda i, nd=nd: (0,) * nd)


_mlp_call = pl.pallas_call(
    _mlp_body,
    grid=(B // BB,),
    in_specs=[
        pl.BlockSpec((BB, DD), lambda i: (i, 0)),
        pl.BlockSpec((BB, NS * EMB), lambda i: (i, 0)),
        pl.BlockSpec((BB, 2 * EMB), lambda i: (i, 0)),
        pl.BlockSpec((BB, 1), lambda i: (i, 0)),
        _full((DD, H)),
        _full((NS * EMB, H)),
        _full((2 * EMB, H)),
        _full((H,)),
        _full((H, H)),
        _full((H,)),
        _full((H, H)),
        _full((H,)),
        _full((1, H)),
        _full((1,)),
        _full((1, DD)),
        _full((1,)),
        _full((1, 1)),
        _full((1,)),
        _full((1, 1)),
        _full((1,)),
    ],
    out_specs=[
        pl.BlockSpec((BB, 1), lambda i: (i, 0)),
        pl.BlockSpec((BB, 1), lambda i: (i, 0)),
    ],
    out_shape=[
        jax.ShapeDtypeStruct((B, 1), jnp.float32),
        jax.ShapeDtypeStruct((B, 1), jnp.float32),
    ],
)


def kernel(sparse_inputs, dense_inputs, seq_inputs_0, seq_inputs_1,
           lin_tables, emb_tables, seq_table_0, seq_table_1,
           W_lin, b_lin, W1, b1, W2, b2, W3, b3, W4, b4, Wf, bf, Wl, bl):
    sp = sparse_inputs.astype(jnp.int32)
    lin_flat = lin_tables.reshape(NS * V)
    sp_flat = sp.reshape(B * NS)
    spt = jnp.pad(sp.T.reshape(NS, B // SPW, SPW).transpose(1, 0, 2),
                  ((0, 0), (0, NSP - NS), (0, 0))).reshape(
        (B // SPW) * NSP, SPW)
    s0r = seq_inputs_0.astype(jnp.int32).reshape(B // 2, 2 * SEQ)
    s1r = seq_inputs_1.astype(jnp.int32).reshape(B // 2, 2 * SEQ)
    fidx = (jnp.arange(WEPC, dtype=jnp.int32) % NS)

    seqout, linout = _make_sl_call()(
        lin_flat, seq_table_0, seq_table_1, spt, s0r, s1r)
    (embout,) = _make_se_call()(emb_tables, sp_flat, fidx)

    W1d = W1[:DD]
    W1e = W1[DD:DD + NS * EMB]
    W1s = W1[DD + NS * EMB:] * jnp.float32(1.0 / SEQ)
    fin, like = _mlp_call(
        dense_inputs, embout.reshape(B, NS * EMB), seqout,
        linout.reshape(B, 1),
        W1d, W1e, W1s, b1, W2, b2, W3, b3,
        W4.reshape(1, H), b4, W_lin.reshape(1, DD), b_lin, Wf, bf, Wl, bl)
    return (fin, like)
